# baseline (speedup 1.0000x reference)
import functools

import jax
import jax.numpy as jnp
from jax import lax
from jax.experimental import pallas as pl
from jax.experimental.pallas import tpu as pltpu
from jax.experimental.pallas import tpu_sc as plsc

ROWS, COLS = 1_000_000, 64
NIDX = 16_384
NC, NS = 2, 16
NW = NC * NS
IDX_CHUNK = 128
CHUNKS_PER_W = NIDX // (NW * IDX_CHUNK)
COPY_CHUNKS = 8
CHUNK_ROWS = ROWS // COPY_CHUNKS


def _copy_body(i_ref, o_ref, *sems):
    copies = [
        pltpu.make_async_copy(
            i_ref.at[pl.ds(c * CHUNK_ROWS, CHUNK_ROWS)],
            o_ref.at[pl.ds(c * CHUNK_ROWS, CHUNK_ROWS)],
            sems[c],
        )
        for c in range(COPY_CHUNKS)
    ]
    for c in copies:
        c.start()
    for c in copies:
        c.wait()


def _tc_copy(inp):
    return pl.pallas_call(
        _copy_body,
        in_specs=[pl.BlockSpec(memory_space=pl.ANY)],
        out_specs=pl.BlockSpec(memory_space=pl.ANY),
        out_shape=jax.ShapeDtypeStruct((ROWS, COLS), jnp.float32),
        scratch_shapes=[pltpu.SemaphoreType.DMA] * COPY_CHUNKS,
    )(inp)


_MESH = plsc.VectorSubcoreMesh(
    core_axis_name="c", subcore_axis_name="s", num_cores=NC, num_subcores=NS
)


@functools.partial(
    pl.kernel,
    mesh=_MESH,
    compiler_params=pltpu.CompilerParams(use_tc_tiling_on_sc=False),
    scratch_types=[
        pltpu.VMEM((CHUNKS_PER_W, IDX_CHUNK), jnp.int32),
        pltpu.VMEM((IDX_CHUNK, COLS), jnp.float32),
        pltpu.SemaphoreType.DMA,
    ],
)
def _sc_zero_rows(idx_hbm, data_ref, idx_v, zeros_v, sem):
    wid = lax.axis_index("s") * NC + lax.axis_index("c")
    pltpu.sync_copy(idx_hbm.at[pl.ds(wid * CHUNKS_PER_W, CHUNKS_PER_W)], idx_v)

    zero16 = jnp.zeros((16,), jnp.float32)

    @pl.loop(0, IDX_CHUNK)
    def _(i):
        for c in range(COLS // 16):
            zeros_v[i, pl.ds(c * 16, 16)] = zero16

    copies = [
        pltpu.make_async_copy(zeros_v, data_ref.at[idx_v.at[j]], sem)
        for j in range(CHUNKS_PER_W)
    ]
    for c in copies:
        c.start()
    for c in copies:
        c.wait()


def kernel(inp, indices):
    copied = _tc_copy(inp)
    ref = jax.new_ref(copied)
    _sc_zero_rows(indices.reshape(NW * CHUNKS_PER_W, IDX_CHUNK), ref)
    return jax.freeze(ref)


# speedup vs baseline: 13.8842x; 13.8842x over previous
import functools

import jax
import jax.numpy as jnp
from jax import lax
from jax.experimental import pallas as pl
from jax.experimental.pallas import tpu as pltpu
from jax.experimental.pallas import tpu_sc as plsc

ROWS, COLS = 1_000_000, 64
NIDX = 16_384
NC, NS = 2, 16
NW = NC * NS
IDX_CHUNK = 128
CHUNKS_PER_W = NIDX // (NW * IDX_CHUNK)


_MESH = plsc.VectorSubcoreMesh(
    core_axis_name="c", subcore_axis_name="s", num_cores=NC, num_subcores=NS
)


@functools.partial(
    pl.kernel,
    mesh=_MESH,
    compiler_params=pltpu.CompilerParams(use_tc_tiling_on_sc=False),
    scratch_types=[
        pltpu.VMEM((CHUNKS_PER_W, IDX_CHUNK), jnp.int32),
        pltpu.VMEM((IDX_CHUNK, COLS), jnp.float32),
        pltpu.SemaphoreType.DMA,
    ],
)
def _sc_zero_rows(idx_hbm, data_ref, idx_v, zeros_v, sem):
    wid = lax.axis_index("s") * NC + lax.axis_index("c")
    pltpu.sync_copy(idx_hbm.at[pl.ds(wid * CHUNKS_PER_W, CHUNKS_PER_W)], idx_v)

    zero16 = jnp.zeros((16,), jnp.float32)

    @pl.loop(0, IDX_CHUNK)
    def _(i):
        for c in range(COLS // 16):
            zeros_v[i, pl.ds(c * 16, 16)] = zero16

    copies = [
        pltpu.make_async_copy(zeros_v, data_ref.at[idx_v.at[j]], sem)
        for j in range(CHUNKS_PER_W)
    ]
    for c in copies:
        c.start()
    for c in copies:
        c.wait()


def kernel(inp, indices):
    ref = jax.new_ref(inp)
    _sc_zero_rows(indices.reshape(NW * CHUNKS_PER_W, IDX_CHUNK), ref)
    return jax.freeze(ref)
